# baseline (speedup 1.0000x reference)
import functools

import jax
import jax.numpy as jnp
from jax import lax
from jax.experimental import pallas as pl
from jax.experimental.pallas import tpu as pltpu
from jax.experimental.pallas import tpu_sc as plsc

N_TOKENS = 4096
D_MODEL = 2048
CONN = 72
CONN_PAD = 128
TOP_K = 2
TBLK = 512

NUM_WORKERS = 32
TPW = N_TOKENS // NUM_WORKERS
LANES = 16
GROUPS = TPW // LANES
TBL_STRIDE = 4
NEG = -1e30


def _logits_body(x_ref, w_ref, out_ref, xout_ref):
    out_ref[:, :CONN] = jnp.dot(x_ref[...], w_ref[...],
                                preferred_element_type=jnp.float32)
    xout_ref[...] = x_ref[...]


@functools.cache
def _compute_logits():
    return pl.pallas_call(
        _logits_body,
        grid=(N_TOKENS // TBLK,),
        in_specs=[
            pl.BlockSpec((TBLK, D_MODEL), lambda i: (i, 0)),
            pl.BlockSpec((D_MODEL, CONN), lambda i: (0, 0)),
        ],
        out_specs=[
            pl.BlockSpec((TBLK, CONN_PAD), lambda i: (i, 0)),
            pl.BlockSpec((TBLK, D_MODEL), lambda i: (i, 0)),
        ],
        out_shape=[
            jax.ShapeDtypeStruct((N_TOKENS, CONN_PAD), jnp.float32),
            jax.ShapeDtypeStruct((N_TOKENS, D_MODEL), jnp.float32),
        ],
    )


def _route_body(logits_hbm, table_hbm, probs_hbm, coords_hbm,
                lg_v, tb_v, pr_v, co_v):
    wid = lax.axis_index("s") * 2 + lax.axis_index("c")
    base = wid * TPW
    pltpu.sync_copy(logits_hbm.at[pl.ds(base, TPW)], lg_v)
    pltpu.sync_copy(table_hbm, tb_v)

    lanes = lax.iota(jnp.int32, 16)
    zeros = jnp.zeros((16,), jnp.int32)
    IL = 4
    for blk in range(GROUPS // IL):
        toks = [(blk * IL + g) * LANES + lanes for g in range(IL)]

        def top2_step(c, carry):
            cv = zeros + c
            outs = []
            for g in range(IL):
                v1, i1, v2, i2, d = carry[5 * g: 5 * g + 5]
                lv = plsc.load_gather(lg_v, [toks[g], cv])
                gt1 = lv > v1
                lo = jnp.minimum(lv, v1)
                v1n = jnp.maximum(lv, v1)
                gt2 = lo > v2
                v2n = jnp.maximum(lo, v2)
                i1n = jnp.where(gt1, cv, i1)
                ilo = jnp.where(gt1, i1, cv)
                i2n = jnp.where(gt2, ilo, i2)
                dn = d * jnp.exp(v1 - v1n) + jnp.exp(lv - v1n)
                outs += [v1n, i1n, v2n, i2n, dn]
            return tuple(outs)

        neg = jnp.full((16,), NEG, jnp.float32)
        zf = jnp.zeros((16,), jnp.float32)
        init = tuple(x for _ in range(IL)
                     for x in (neg, zeros, neg, zeros, zf))
        top2 = lax.fori_loop(0, CONN, top2_step, init)

        for g in range(IL):
            v1, i1, v2, i2, d = top2[5 * g: 5 * g + 5]
            tok = toks[g]
            inv_d = 1.0 / d
            p1 = inv_d
            p2 = jnp.exp(v2 - v1) * inv_d
            plsc.store_scatter(pr_v, [tok, zeros], p1)
            plsc.store_scatter(pr_v, [tok, zeros + 1], p2)
            for comp in range(3):
                c1 = plsc.load_gather(tb_v, [i1 * TBL_STRIDE + comp])
                c2 = plsc.load_gather(tb_v, [i2 * TBL_STRIDE + comp])
                plsc.store_scatter(co_v, [tok, zeros, zeros + comp], c1)
                plsc.store_scatter(co_v, [tok, zeros + 1, zeros + comp], c2)

    pltpu.sync_copy(pr_v, probs_hbm.at[pl.ds(base, TPW)])
    pltpu.sync_copy(co_v, coords_hbm.at[pl.ds(base, TPW)])


@functools.cache
def _route():
    return pl.kernel(
        _route_body,
        out_type=(
            jax.ShapeDtypeStruct((N_TOKENS, TOP_K), jnp.float32),
            jax.ShapeDtypeStruct((N_TOKENS, TOP_K, 3), jnp.int32),
        ),
        mesh=plsc.VectorSubcoreMesh(core_axis_name="c", subcore_axis_name="s"),
        compiler_params=pltpu.CompilerParams(needs_layout_passes=False),
        scratch_types=[
            pltpu.VMEM((TPW, CONN_PAD), jnp.float32),
            pltpu.VMEM((CONN * TBL_STRIDE,), jnp.int32),
            pltpu.VMEM((TPW, TOP_K), jnp.float32),
            pltpu.VMEM((TPW, TOP_K, 3), jnp.int32),
        ],
    )


def kernel(input, W_router, neuron_connections):
    table = jnp.pad(neuron_connections, ((0, 0), (0, TBL_STRIDE - 3))).reshape(-1)
    logits, x_copy = _compute_logits()(input, W_router)
    top_probs, selected = _route()(logits, table)
    return (x_copy, top_probs, selected)


# speedup vs baseline: 1.0342x; 1.0342x over previous
import functools

import jax
import jax.numpy as jnp
from jax import lax
from jax.experimental import pallas as pl
from jax.experimental.pallas import tpu as pltpu
from jax.experimental.pallas import tpu_sc as plsc

N_TOKENS = 4096
D_MODEL = 2048
CONN = 72
CONN_PAD = 128
TOP_K = 2
TBLK = 1024

NUM_WORKERS = 32
TPW = N_TOKENS // NUM_WORKERS
LANES = 16
GROUPS = TPW // LANES
TBL_STRIDE = 4
NEG = -1e30


def _logits_body(x_ref, w_ref, out_ref, xout_ref):
    out_ref[:, :CONN] = jnp.dot(x_ref[...], w_ref[...],
                                preferred_element_type=jnp.float32)
    xout_ref[...] = x_ref[...]


@functools.cache
def _compute_logits():
    return pl.pallas_call(
        _logits_body,
        grid=(N_TOKENS // TBLK,),
        in_specs=[
            pl.BlockSpec((TBLK, D_MODEL), lambda i: (i, 0)),
            pl.BlockSpec((D_MODEL, CONN), lambda i: (0, 0)),
        ],
        out_specs=[
            pl.BlockSpec((TBLK, CONN_PAD), lambda i: (i, 0)),
            pl.BlockSpec((TBLK, D_MODEL), lambda i: (i, 0)),
        ],
        out_shape=[
            jax.ShapeDtypeStruct((N_TOKENS, CONN_PAD), jnp.float32),
            jax.ShapeDtypeStruct((N_TOKENS, D_MODEL), jnp.float32),
        ],
    )


def _route_body(logits_hbm, table_hbm, probs_hbm, coords_hbm,
                lg_a, lg_b, tb_v, pr_v, co_v, sem_a, sem_b):
    wid = lax.axis_index("s") * 2 + lax.axis_index("c")
    base = wid * TPW
    HALF = TPW // 2
    cp_a = pltpu.async_copy(logits_hbm.at[pl.ds(base, HALF)], lg_a, sem_a)
    cp_b = pltpu.async_copy(logits_hbm.at[pl.ds(base + HALF, HALF)], lg_b,
                            sem_b)
    pltpu.sync_copy(table_hbm, tb_v)

    lanes = lax.iota(jnp.int32, 16)
    zeros = jnp.zeros((16,), jnp.int32)
    IL = 4
    for blk in range(GROUPS // IL):
        lg_v = (lg_a, lg_b)[blk]
        (cp_a, cp_b)[blk].wait()
        loc = [g * LANES + lanes for g in range(IL)]
        toks = [blk * HALF + l for l in loc]

        def top2_step(c, carry):
            cv = zeros + c
            outs = []
            for g in range(IL):
                v1, i1, v2, i2, d = carry[5 * g: 5 * g + 5]
                lv = plsc.load_gather(lg_v, [loc[g], cv])
                gt1 = lv > v1
                lo = jnp.minimum(lv, v1)
                v1n = jnp.maximum(lv, v1)
                gt2 = lo > v2
                v2n = jnp.maximum(lo, v2)
                i1n = jnp.where(gt1, cv, i1)
                ilo = jnp.where(gt1, i1, cv)
                i2n = jnp.where(gt2, ilo, i2)
                dn = d * jnp.exp(v1 - v1n) + jnp.exp(lv - v1n)
                outs += [v1n, i1n, v2n, i2n, dn]
            return tuple(outs)

        neg = jnp.full((16,), NEG, jnp.float32)
        zf = jnp.zeros((16,), jnp.float32)
        init = tuple(x for _ in range(IL)
                     for x in (neg, zeros, neg, zeros, zf))
        top2 = lax.fori_loop(0, CONN, top2_step, init)

        for g in range(IL):
            v1, i1, v2, i2, d = top2[5 * g: 5 * g + 5]
            tok = toks[g]
            inv_d = 1.0 / d
            p1 = inv_d
            p2 = jnp.exp(v2 - v1) * inv_d
            plsc.store_scatter(pr_v, [tok, zeros], p1)
            plsc.store_scatter(pr_v, [tok, zeros + 1], p2)
            for comp in range(3):
                c1 = plsc.load_gather(tb_v, [i1 * TBL_STRIDE + comp])
                c2 = plsc.load_gather(tb_v, [i2 * TBL_STRIDE + comp])
                plsc.store_scatter(co_v, [tok, zeros, zeros + comp], c1)
                plsc.store_scatter(co_v, [tok, zeros + 1, zeros + comp], c2)

    pltpu.sync_copy(pr_v, probs_hbm.at[pl.ds(base, TPW)])
    pltpu.sync_copy(co_v, coords_hbm.at[pl.ds(base, TPW)])


@functools.cache
def _route():
    return pl.kernel(
        _route_body,
        out_type=(
            jax.ShapeDtypeStruct((N_TOKENS, TOP_K), jnp.float32),
            jax.ShapeDtypeStruct((N_TOKENS, TOP_K, 3), jnp.int32),
        ),
        mesh=plsc.VectorSubcoreMesh(core_axis_name="c", subcore_axis_name="s"),
        compiler_params=pltpu.CompilerParams(needs_layout_passes=False),
        scratch_types=[
            pltpu.VMEM((TPW // 2, CONN_PAD), jnp.float32),
            pltpu.VMEM((TPW // 2, CONN_PAD), jnp.float32),
            pltpu.VMEM((CONN * TBL_STRIDE,), jnp.int32),
            pltpu.VMEM((TPW, TOP_K), jnp.float32),
            pltpu.VMEM((TPW, TOP_K, 3), jnp.int32),
            pltpu.SemaphoreType.DMA,
            pltpu.SemaphoreType.DMA,
        ],
    )


def kernel(input, W_router, neuron_connections):
    table = jnp.pad(neuron_connections, ((0, 0), (0, TBL_STRIDE - 3))).reshape(-1)
    logits, x_copy = _compute_logits()(input, W_router)
    top_probs, selected = _route()(logits, table)
    return (x_copy, top_probs, selected)
